# baseline (speedup 1.0000x reference)
import functools

import jax
import jax.numpy as jnp
from jax import lax
from jax.experimental import pallas as pl
from jax.experimental.pallas import tpu as pltpu
from jax.experimental.pallas import tpu_sc as plsc

N_ROWS = 100000
EMBED = 16
INPUT = 14
OUTPUT = 1
BATCH = 16384

_info = plsc.get_sparse_core_info()
_NC, _NS = _info.num_cores, _info.num_subcores
_NW = _NC * _NS
_B_PER_W = BATCH // _NW

_CW = 6400
_NCHUNK = 16
_ALIGNED = 99968
_TAIL_BLK = 781
_T_PAD = _CW * _NCHUNK


def _wcol(w_ref, offset, n):
    k = lax.broadcasted_iota(jnp.int32, (n, 1), 0)
    col = jnp.zeros((n, 1), jnp.float32)
    for i in range(n):
        col = jnp.where(k == i, w_ref[0, offset + i], col)
    return col


def _chunk_w(i):
    return min(_CW, _ALIGNED - i * _CW)


def _tc_table_body(tt_hbm, w_ref, b_ref, ttail_ref, t_ref, bufs, sems):
    wecol = _wcol(w_ref, INPUT, EMBED)

    def copy(i):
        w = _chunk_w(i)
        return pltpu.make_async_copy(
            tt_hbm.at[:, pl.ds(i * _CW, w)],
            bufs.at[i % _NBUF, :, pl.ds(0, w)],
            sems.at[i % _NBUF],
        )

    for i in range(_NBUF):
        copy(i).start()
    for i in range(_NCHUNK):
        copy(i).wait()
        if i + _NBUF < _NCHUNK:
            copy(i + _NBUF).start()
        v = bufs[i % _NBUF]
        t_ref[pl.ds(i * _CW, _CW)] = jnp.sum(v * wecol, axis=0) + b_ref[0]
    t_ref[pl.ds(_ALIGNED, 128)] = jnp.sum(ttail_ref[...] * wecol, axis=0) + b_ref[0]


_NBUF = 16


def _tc_table(tableT, WT, b):
    return pl.pallas_call(
        _tc_table_body,
        grid=(1,),
        in_specs=[
            pl.BlockSpec(memory_space=pl.ANY),
            pl.BlockSpec(memory_space=pltpu.SMEM),
            pl.BlockSpec(memory_space=pltpu.SMEM),
            pl.BlockSpec((EMBED, 128), lambda i: (0, _TAIL_BLK)),
        ],
        out_specs=pl.BlockSpec((_T_PAD,), lambda i: (0,)),
        out_shape=jax.ShapeDtypeStruct((_T_PAD,), jnp.float32),
        scratch_shapes=[
            pltpu.VMEM((_NBUF, EMBED, _CW), jnp.float32),
            pltpu.SemaphoreType.DMA((_NBUF,)),
        ],
    )(tableT, WT, b, tableT)


def _tc_x_body(xt_ref, w_ref, xw_ref):
    wxcol = _wcol(w_ref, 0, INPUT)
    xw_ref[...] = jnp.sum(xt_ref[...] * wxcol, axis=0)


def _tc_x(xT, WT):
    return pl.pallas_call(
        _tc_x_body,
        in_specs=[
            pl.BlockSpec((INPUT, BATCH), lambda: (0, 0)),
            pl.BlockSpec(memory_space=pltpu.SMEM),
        ],
        out_specs=pl.BlockSpec((BATCH,), lambda: (0,)),
        out_shape=jax.ShapeDtypeStruct((BATCH,), jnp.float32),
    )(xT, WT)


def _sc_body(t_hbm, c_hbm, out_hbm, idx_v, tg_v, sem):
    wid = lax.axis_index("s") * _NC + lax.axis_index("c")
    base = wid * _B_PER_W
    pltpu.sync_copy(c_hbm.at[pl.ds(base, _B_PER_W)], idx_v)
    pltpu.async_copy(t_hbm.at[idx_v], tg_v, sem).wait()
    pltpu.sync_copy(tg_v, out_hbm.at[pl.ds(base, _B_PER_W)])


_sc_gather = pl.kernel(
    _sc_body,
    out_type=jax.ShapeDtypeStruct((BATCH,), jnp.float32),
    mesh=plsc.VectorSubcoreMesh(core_axis_name="c", subcore_axis_name="s"),
    scratch_types=[
        pltpu.VMEM((_B_PER_W,), jnp.int32),
        pltpu.VMEM((_B_PER_W,), jnp.float32),
        pltpu.SemaphoreType.DMA,
    ],
    compiler_params=pltpu.CompilerParams(use_tc_tiling_on_sc=False),
)


def _tc_add_body(a_ref, b_ref, o_ref):
    o_ref[...] = a_ref[...] + b_ref[...]


def _tc_add(a, b2):
    return pl.pallas_call(
        _tc_add_body,
        out_shape=jax.ShapeDtypeStruct((BATCH,), jnp.float32),
    )(a, b2)


@jax.jit
def kernel(x, c, table, W, b):
    tableT = table.T
    xT = x.T
    WT = W.T
    t = _tc_table(tableT, WT, b)
    tg = _sc_gather(t, c.astype(jnp.int32))
    xw = _tc_x(xT, WT)
    out = _tc_add(tg, xw)
    return out.reshape(BATCH, OUTPUT)


# speedup vs baseline: 1.0094x; 1.0094x over previous
import functools

import jax
import jax.numpy as jnp
from jax import lax
from jax.experimental import pallas as pl
from jax.experimental.pallas import tpu as pltpu
from jax.experimental.pallas import tpu_sc as plsc

N_ROWS = 100000
EMBED = 16
INPUT = 14
OUTPUT = 1
BATCH = 16384

_info = plsc.get_sparse_core_info()
_NC, _NS = _info.num_cores, _info.num_subcores
_NW = _NC * _NS
_B_PER_W = BATCH // _NW

_CW = 6400
_NCHUNK = 16
_ALIGNED = 99968
_TAIL_BLK = 781
_T_PAD = _CW * _NCHUNK


def _wcol(w_ref, offset, n):
    k = lax.broadcasted_iota(jnp.int32, (n, 1), 0)
    col = jnp.zeros((n, 1), jnp.float32)
    for i in range(n):
        col = jnp.where(k == i, w_ref[0, offset + i], col)
    return col


def _chunk_w(i):
    return min(_CW, _ALIGNED - i * _CW)


def _tc_table_body(tt_hbm, w_ref, b_ref, ttail_ref, t_ref, bufs, sems):
    wecol = _wcol(w_ref, INPUT, EMBED)

    def copy(i):
        w = _chunk_w(i)
        return pltpu.make_async_copy(
            tt_hbm.at[:, pl.ds(i * _CW, w)],
            bufs.at[i % _NBUF, :, pl.ds(0, w)],
            sems.at[i % _NBUF],
        )

    for i in range(_NBUF):
        copy(i).start()
    for i in range(_NCHUNK):
        copy(i).wait()
        if i + _NBUF < _NCHUNK:
            copy(i + _NBUF).start()
        v = bufs[i % _NBUF]
        t_ref[pl.ds(i * _CW, _CW)] = jnp.sum(v * wecol, axis=0) + b_ref[0]
    t_ref[pl.ds(_ALIGNED, 128)] = jnp.sum(ttail_ref[...] * wecol, axis=0) + b_ref[0]


_NBUF = 16


def _tc_table(tableT, WT, b):
    return pl.pallas_call(
        _tc_table_body,
        grid=(1,),
        in_specs=[
            pl.BlockSpec(memory_space=pl.ANY),
            pl.BlockSpec(memory_space=pltpu.SMEM),
            pl.BlockSpec(memory_space=pltpu.SMEM),
            pl.BlockSpec((EMBED, 128), lambda i: (0, _TAIL_BLK)),
        ],
        out_specs=pl.BlockSpec((_T_PAD,), lambda i: (0,)),
        out_shape=jax.ShapeDtypeStruct((_T_PAD,), jnp.float32),
        scratch_shapes=[
            pltpu.VMEM((_NBUF, EMBED, _CW), jnp.float32),
            pltpu.SemaphoreType.DMA((_NBUF,)),
        ],
    )(tableT, WT, b, tableT)


def _tc_x_body(xt_ref, w_ref, xw_ref):
    wxcol = _wcol(w_ref, 0, INPUT)
    xw_ref[...] = jnp.sum(xt_ref[...] * wxcol, axis=0)


def _tc_x(xT, WT):
    return pl.pallas_call(
        _tc_x_body,
        in_specs=[
            pl.BlockSpec((INPUT, BATCH), lambda: (0, 0)),
            pl.BlockSpec(memory_space=pltpu.SMEM),
        ],
        out_specs=pl.BlockSpec((BATCH,), lambda: (0,)),
        out_shape=jax.ShapeDtypeStruct((BATCH,), jnp.float32),
    )(xT, WT)


def _sc_body(t_hbm, c_hbm, out_hbm, idx_v, tg_v, sems):
    wid = lax.axis_index("s") * _NC + lax.axis_index("c")
    base = wid * _B_PER_W
    pltpu.sync_copy(c_hbm.at[pl.ds(base, _B_PER_W)], idx_v)
    q = _B_PER_W // 4
    copies = [
        pltpu.async_copy(
            t_hbm.at[idx_v.at[pl.ds(j * q, q)]],
            tg_v.at[pl.ds(j * q, q)],
            sems.at[j],
        )
        for j in range(4)
    ]
    for cp in copies:
        cp.wait()
    pltpu.sync_copy(tg_v, out_hbm.at[pl.ds(base, _B_PER_W)])


_sc_gather = pl.kernel(
    _sc_body,
    out_type=jax.ShapeDtypeStruct((BATCH,), jnp.float32),
    mesh=plsc.VectorSubcoreMesh(core_axis_name="c", subcore_axis_name="s"),
    scratch_types=[
        pltpu.VMEM((_B_PER_W,), jnp.int32),
        pltpu.VMEM((_B_PER_W,), jnp.float32),
        pltpu.SemaphoreType.DMA((4,)),
    ],
    compiler_params=pltpu.CompilerParams(use_tc_tiling_on_sc=False),
)


def _tc_add_body(a_ref, b_ref, o_ref):
    o_ref[...] = a_ref[...] + b_ref[...]


def _tc_add(a, b2):
    return pl.pallas_call(
        _tc_add_body,
        out_shape=jax.ShapeDtypeStruct((BATCH,), jnp.float32),
    )(a, b2)


@jax.jit
def kernel(x, c, table, W, b):
    tableT = table.T
    xT = x.T
    WT = W.T
    t = _tc_table(tableT, WT, b)
    tg = _sc_gather(t, c.astype(jnp.int32))
    xw = _tc_x(xT, WT)
    out = _tc_add(tg, xw)
    return out.reshape(BATCH, OUTPUT)
